# baseline (speedup 1.0000x reference)
import functools

import jax
import jax.numpy as jnp
from jax import lax
from jax.experimental import pallas as pl
from jax.experimental.pallas import tpu as pltpu
from jax.experimental.pallas import tpu_sc as plsc

NC = 2
NS = 16
L = 16
KB = 128


def _sc_deg(col, ew, n):
    e = col.shape[0]
    per_w = e // (NC * NS)
    nfull = per_w // KB
    rem = per_w - nfull * KB
    rows_per = (n // NS) // 8 * 8
    rows_tail = n - NS * rows_per

    mesh = plsc.VectorSubcoreMesh(core_axis_name="c", subcore_axis_name="s",
                                  num_cores=NC, num_subcores=NS)

    @functools.partial(
        pl.kernel,
        out_type=jax.ShapeDtypeStruct((NC, n, L), jnp.float32),
        mesh=mesh,
        compiler_params=pltpu.CompilerParams(use_tc_tiling_on_sc=False),
        scratch_types=[
            pltpu.VMEM_SHARED((n, L), jnp.float32),
            pltpu.VMEM((KB, L), jnp.float32),
            pltpu.VMEM((KB,), jnp.int32),
            pltpu.VMEM((KB,), jnp.float32),
            pltpu.VMEM((L, L), jnp.float32),
            pltpu.VMEM((L,), jnp.int32),
            pltpu.VMEM((L,), jnp.float32),
        ],
    )
    def k(col_hbm, ew_hbm, out_hbm, acc, msg, cbuf, ebuf, msg_t, cbuf_t, ebuf_t):
        c = lax.axis_index("c")
        s = lax.axis_index("s")
        zero = jnp.zeros((L,), jnp.float32)

        def zrow(kk, _):
            msg[kk, :] = zero
            return 0
        lax.fori_loop(0, KB, zrow, 0)

        base = s * rows_per
        nz = rows_per // KB
        for i in range(nz):
            pltpu.sync_copy(msg, acc.at[pl.ds(base + i * KB, KB)])
        tz = rows_per - nz * KB
        if tz:
            pltpu.sync_copy(msg.at[pl.ds(0, tz)], acc.at[pl.ds(base + nz * KB, tz)])
        if rows_tail:
            @pl.when(s == NS - 1)
            def _():
                pltpu.sync_copy(msg.at[pl.ds(0, rows_tail)],
                                acc.at[pl.ds(NS * rows_per, rows_tail)])
        plsc.subcore_barrier()

        wbase = (c * NS + s) * per_w

        def fill(nb, ebuf_, msg_):
            def body(g, _):
                ew16 = ebuf_[pl.ds(g * L, L)]
                for jj in range(L):
                    msg_[g * L + jj, :] = jnp.full((L,), ew16[jj], jnp.float32)
                return 0
            lax.fori_loop(0, nb // L, body, 0)

        def batch(ib, _):
            eb = wbase + ib * KB
            pltpu.sync_copy(col_hbm.at[pl.ds(eb, KB)], cbuf)
            pltpu.sync_copy(ew_hbm.at[pl.ds(eb, KB)], ebuf)
            fill(KB, ebuf, msg)
            pltpu.sync_copy(msg, acc.at[cbuf], add=True)
            return 0
        lax.fori_loop(0, nfull, batch, 0)

        if rem:
            eb = wbase + nfull * KB
            pltpu.sync_copy(col_hbm.at[pl.ds(eb, rem)], cbuf_t.at[pl.ds(0, rem)])
            pltpu.sync_copy(ew_hbm.at[pl.ds(eb, rem)], ebuf_t.at[pl.ds(0, rem)])
            fill(rem, ebuf_t, msg_t)
            pltpu.sync_copy(msg_t.at[pl.ds(0, rem)], acc.at[cbuf_t], add=True)

        plsc.subcore_barrier()
        pltpu.sync_copy(acc.at[pl.ds(base, rows_per)],
                        out_hbm.at[c, pl.ds(base, rows_per)])
        if rows_tail:
            @pl.when(s == NS - 1)
            def _():
                pltpu.sync_copy(acc.at[pl.ds(NS * rows_per, rows_tail)],
                                out_hbm.at[c, pl.ds(NS * rows_per, rows_tail)])

    return k(col, ew)


def _sc_aggregate(xwp, row, col2d, ew, n, d):
    e = row.shape[0]
    kb2 = KB
    per_w = e // (NC * NS)
    nfull = per_w // kb2
    assert per_w == nfull * kb2
    rows_per = (n // NS) // 8 * 8
    rows_tail = n - NS * rows_per
    nd = d // L

    mesh = plsc.VectorSubcoreMesh(core_axis_name="c", subcore_axis_name="s",
                                  num_cores=NC, num_subcores=NS)

    @functools.partial(
        pl.kernel,
        out_type=jax.ShapeDtypeStruct((NC, n, d), jnp.float32),
        mesh=mesh,
        compiler_params=pltpu.CompilerParams(use_tc_tiling_on_sc=False),
        scratch_types=[
            pltpu.VMEM_SHARED((n, d), jnp.float32),
            pltpu.VMEM((2, kb2, d), jnp.float32),
            pltpu.VMEM((3, kb2), jnp.int32),
            pltpu.VMEM((3, 1, KB), jnp.int32),
            pltpu.VMEM((3, kb2), jnp.float32),
            pltpu.SemaphoreType.DMA((3,)),
            pltpu.SemaphoreType.DMA((2,)),
            pltpu.SemaphoreType.DMA((2,)),
        ],
    )
    def k(xwp_hbm, row_hbm, col2_hbm, ew_hbm, out_hbm,
          acc, msg, rbuf, cbuf, ebuf, sem_i, sem_g, sem_s):
        c = lax.axis_index("c")
        s = lax.axis_index("s")
        zero = jnp.zeros((L,), jnp.float32)

        def zrow(kk, _):
            for dd in range(nd):
                msg[0, kk, pl.ds(dd * L, L)] = zero
            return 0
        lax.fori_loop(0, kb2, zrow, 0)

        base = s * rows_per
        nz = rows_per // kb2
        for i in range(nz):
            pltpu.sync_copy(msg.at[0], acc.at[pl.ds(base + i * kb2, kb2)])
        tz = rows_per - nz * kb2
        if tz:
            pltpu.sync_copy(msg.at[0, pl.ds(0, tz)],
                            acc.at[pl.ds(base + nz * kb2, tz)])
        if rows_tail:
            @pl.when(s == NS - 1)
            def _():
                pltpu.sync_copy(msg.at[0, pl.ds(0, rows_tail)],
                                acc.at[pl.ds(NS * rows_per, rows_tail)])
        plsc.subcore_barrier()

        wbase = (c * NS + s) * per_w
        wrow = wbase // KB

        def idx_copies(t, slot):
            eb = wbase + t * kb2
            return (pltpu.make_async_copy(row_hbm.at[pl.ds(eb, kb2)],
                                          rbuf.at[slot], sem_i.at[slot]),
                    pltpu.make_async_copy(col2_hbm.at[pl.ds(wrow + t, 1)],
                                          cbuf.at[slot], sem_i.at[slot]),
                    pltpu.make_async_copy(ew_hbm.at[pl.ds(eb, kb2)],
                                          ebuf.at[slot], sem_i.at[slot]))

        def gather_copies(slot3, slot2):
            return tuple(
                pltpu.make_async_copy(
                    xwp_hbm.at[rbuf.at[slot3, pl.ds(j * KB, KB)]],
                    msg.at[slot2, pl.ds(j * KB, KB)], sem_g.at[slot2])
                for j in range(1))

        def scatter_copies(slot3, slot2):
            return tuple(
                pltpu.make_async_copy(
                    msg.at[slot2, pl.ds(j * KB, KB)],
                    acc.at[cbuf.at[slot3, j]], sem_s.at[slot2])
                for j in range(1))

        def scale(slot3, slot2):
            def body(g, _):
                ew16 = ebuf[slot3, pl.ds(g * L, L)]
                for jj in range(L):
                    kk = g * L + jj
                    ewv = jnp.full((L,), ew16[jj], jnp.float32)
                    for dd in range(nd):
                        msg[slot2, kk, pl.ds(dd * L, L)] = \
                            msg[slot2, kk, pl.ds(dd * L, L)] * ewv
                return 0
            lax.fori_loop(0, kb2 // L, body, 0)

        if nfull > 0:
            for d_ in idx_copies(0, 0):
                d_.start()
            if nfull > 1:
                for d_ in idx_copies(1, 1):
                    d_.start()
            for d_ in idx_copies(0, 0):
                d_.wait()
            for d_ in gather_copies(0, 0):
                d_.start()

            def pipe(i, _):
                s2 = i % 2
                s2n = (i + 1) % 2
                s3 = i % 3
                s3n = (i + 1) % 3

                @pl.when(i + 1 < nfull)
                def _():
                    for d_ in idx_copies(i + 1, s3n):
                        d_.wait()

                @pl.when(i >= 1)
                def _():
                    for d_ in scatter_copies((i - 1) % 3, s2n):
                        d_.wait()

                @pl.when(i + 1 < nfull)
                def _():
                    for d_ in gather_copies(s3n, s2n):
                        d_.start()

                @pl.when(i + 2 < nfull)
                def _():
                    for d_ in idx_copies(i + 2, (i + 2) % 3):
                        d_.start()

                for d_ in gather_copies(s3, s2):
                    d_.wait()
                scale(s3, s2)
                for d_ in scatter_copies(s3, s2):
                    d_.start(add=True)
                return 0
            lax.fori_loop(0, nfull, pipe, 0)

            for d_ in scatter_copies((nfull - 1) % 3, (nfull - 1) % 2):
                d_.wait()

        plsc.subcore_barrier()
        pltpu.sync_copy(acc.at[pl.ds(base, rows_per)],
                        out_hbm.at[c, pl.ds(base, rows_per)])
        if rows_tail:
            @pl.when(s == NS - 1)
            def _():
                pltpu.sync_copy(acc.at[pl.ds(NS * rows_per, rows_tail)],
                                out_hbm.at[c, pl.ds(NS * rows_per, rows_tail)])

    return k(xwp, row, col2d, ew)


def _tc_pre(x, w0, degp, n, d):
    def body(x_ref, w_ref, degp_ref, dinv_ref, xwp_ref):
        dsum = jnp.sum(degp_ref[0], axis=1, keepdims=True) \
             + jnp.sum(degp_ref[1], axis=1, keepdims=True)
        deg = 1.0 + dsum * (1.0 / L)
        dinv = lax.rsqrt(deg)
        dinv_ref[...] = dinv
        xw = lax.dot_general(x_ref[...], w_ref[...], (((1,), (1,)), ((), ())),
                             preferred_element_type=jnp.float32)
        xwp_ref[...] = xw * dinv

    return pl.pallas_call(
        body,
        out_shape=(jax.ShapeDtypeStruct((n, 1), jnp.float32),
                   jax.ShapeDtypeStruct((n, d), jnp.float32)),
    )(x, w0, degp)


def _tc_post(sp, xwp, dinv, b, gamma, beta, w_next, n, d):
    has_next = w_next is not None

    def body(*refs):
        if has_next:
            sp_ref, xwp_ref, dinv_ref, b_ref, g_ref, be_ref, wn_ref, out_ref = refs
        else:
            sp_ref, xwp_ref, dinv_ref, b_ref, g_ref, be_ref, out_ref = refs
        dinv = dinv_ref[...]
        pre = dinv * (sp_ref[0] + sp_ref[1] + xwp_ref[...]) + b_ref[...]
        mean = jnp.mean(pre, axis=0, keepdims=True)
        var = jnp.mean((pre - mean) ** 2, axis=0, keepdims=True)
        h = (pre - mean) * lax.rsqrt(var + 1e-5) * g_ref[...] + be_ref[...]
        h = jnp.maximum(h, 0.0)
        if has_next:
            xw = lax.dot_general(h, wn_ref[...], (((1,), (1,)), ((), ())),
                                 preferred_element_type=jnp.float32)
            out_ref[...] = xw * dinv
        else:
            out_ref[...] = h

    args = [sp, xwp, dinv, b, gamma, beta]
    if has_next:
        args.append(w_next)
    return pl.pallas_call(
        body,
        out_shape=jax.ShapeDtypeStruct((n, d), jnp.float32),
    )(*args)


def kernel(node_features, edge_index, edges_weight,
           W0, b0, gamma0, beta0, W1, b1, gamma1, beta1, W2, b2, gamma2, beta2):
    n, d = node_features.shape
    e = edge_index.shape[1]
    block = NC * NS * KB
    ep = -(-e // block) * block
    pad = ep - e
    row = jnp.concatenate([edge_index[0], jnp.zeros((pad,), edge_index.dtype)])
    col = jnp.concatenate([edge_index[1], jnp.zeros((pad,), edge_index.dtype)])
    ew = jnp.concatenate([edges_weight,
                          jnp.zeros((pad,), edges_weight.dtype)])
    col2d = col.reshape(ep // KB, KB)

    degp = _sc_deg(col, ew, n)
    dinv, xwp = _tc_pre(node_features, W0, degp, n, d)

    params = [(b0, gamma0, beta0, W1), (b1, gamma1, beta1, W2),
              (b2, gamma2, beta2, None)]
    for b, g, be, w_next in params:
        sp = _sc_aggregate(xwp, row, col2d, ew, n, d)
        xwp = _tc_post(sp, xwp, dinv, b, g, be, w_next, n, d)
    return xwp


# speedup vs baseline: 1.0220x; 1.0220x over previous
import functools

import jax
import jax.numpy as jnp
from jax import lax
from jax.experimental import pallas as pl
from jax.experimental.pallas import tpu as pltpu
from jax.experimental.pallas import tpu_sc as plsc

NC = 2
NS = 16
L = 16
KB = 128


def _sc_deg(col, ew, n):
    e = col.shape[0]
    per_w = e // (NC * NS)
    nfull = per_w // KB
    rem = per_w - nfull * KB
    rows_per = (n // NS) // 8 * 8
    rows_tail = n - NS * rows_per

    mesh = plsc.VectorSubcoreMesh(core_axis_name="c", subcore_axis_name="s",
                                  num_cores=NC, num_subcores=NS)

    @functools.partial(
        pl.kernel,
        out_type=jax.ShapeDtypeStruct((NC, n, L), jnp.float32),
        mesh=mesh,
        compiler_params=pltpu.CompilerParams(use_tc_tiling_on_sc=False),
        scratch_types=[
            pltpu.VMEM_SHARED((n, L), jnp.float32),
            pltpu.VMEM((KB, L), jnp.float32),
            pltpu.VMEM((KB,), jnp.int32),
            pltpu.VMEM((KB,), jnp.float32),
            pltpu.VMEM((L, L), jnp.float32),
            pltpu.VMEM((L,), jnp.int32),
            pltpu.VMEM((L,), jnp.float32),
        ],
    )
    def k(col_hbm, ew_hbm, out_hbm, acc, msg, cbuf, ebuf, msg_t, cbuf_t, ebuf_t):
        c = lax.axis_index("c")
        s = lax.axis_index("s")
        zero = jnp.zeros((L,), jnp.float32)

        def zrow(kk, _):
            msg[kk, :] = zero
            return 0
        lax.fori_loop(0, KB, zrow, 0)

        base = s * rows_per
        nz = rows_per // KB
        for i in range(nz):
            pltpu.sync_copy(msg, acc.at[pl.ds(base + i * KB, KB)])
        tz = rows_per - nz * KB
        if tz:
            pltpu.sync_copy(msg.at[pl.ds(0, tz)], acc.at[pl.ds(base + nz * KB, tz)])
        if rows_tail:
            @pl.when(s == NS - 1)
            def _():
                pltpu.sync_copy(msg.at[pl.ds(0, rows_tail)],
                                acc.at[pl.ds(NS * rows_per, rows_tail)])
        plsc.subcore_barrier()

        wbase = (c * NS + s) * per_w

        def fill(nb, ebuf_, msg_):
            def body(g, _):
                ew16 = ebuf_[pl.ds(g * L, L)]
                for jj in range(L):
                    msg_[g * L + jj, :] = jnp.full((L,), ew16[jj], jnp.float32)
                return 0
            lax.fori_loop(0, nb // L, body, 0)

        def batch(ib, _):
            eb = wbase + ib * KB
            pltpu.sync_copy(col_hbm.at[pl.ds(eb, KB)], cbuf)
            pltpu.sync_copy(ew_hbm.at[pl.ds(eb, KB)], ebuf)
            fill(KB, ebuf, msg)
            pltpu.sync_copy(msg, acc.at[cbuf], add=True)
            return 0
        lax.fori_loop(0, nfull, batch, 0)

        if rem:
            eb = wbase + nfull * KB
            pltpu.sync_copy(col_hbm.at[pl.ds(eb, rem)], cbuf_t.at[pl.ds(0, rem)])
            pltpu.sync_copy(ew_hbm.at[pl.ds(eb, rem)], ebuf_t.at[pl.ds(0, rem)])
            fill(rem, ebuf_t, msg_t)
            pltpu.sync_copy(msg_t.at[pl.ds(0, rem)], acc.at[cbuf_t], add=True)

        plsc.subcore_barrier()
        pltpu.sync_copy(acc.at[pl.ds(base, rows_per)],
                        out_hbm.at[c, pl.ds(base, rows_per)])
        if rows_tail:
            @pl.when(s == NS - 1)
            def _():
                pltpu.sync_copy(acc.at[pl.ds(NS * rows_per, rows_tail)],
                                out_hbm.at[c, pl.ds(NS * rows_per, rows_tail)])

    return k(col, ew)


def _sc_aggregate(xwp, row, col2d, ew, n, d):
    e = row.shape[0]
    per_w = e // (NC * NS)
    nfull = per_w // KB
    assert per_w == nfull * KB and nfull % 2 == 0
    npairs = nfull // 2
    rows_per = (n // NS) // 8 * 8
    rows_tail = n - NS * rows_per
    nd = d // L

    mesh = plsc.VectorSubcoreMesh(core_axis_name="c", subcore_axis_name="s",
                                  num_cores=NC, num_subcores=NS)

    @functools.partial(
        pl.kernel,
        out_type=jax.ShapeDtypeStruct((NC, n, d), jnp.float32),
        mesh=mesh,
        compiler_params=pltpu.CompilerParams(use_tc_tiling_on_sc=False),
        scratch_types=[
            pltpu.VMEM_SHARED((n, d), jnp.float32),
            pltpu.VMEM((KB, d), jnp.float32),
            pltpu.VMEM((KB, d), jnp.float32),
            pltpu.VMEM((2, KB), jnp.int32),
            pltpu.VMEM((2, 1, KB), jnp.int32),
            pltpu.VMEM((2, KB), jnp.float32),
            pltpu.SemaphoreType.DMA((2,)),
            pltpu.SemaphoreType.DMA((2,)),
            pltpu.SemaphoreType.DMA((2,)),
            pltpu.SemaphoreType.DMA((2,)),
            pltpu.SemaphoreType.DMA((2,)),
        ],
    )
    def k(xwp_hbm, row_hbm, col2_hbm, ew_hbm, out_hbm,
          acc, msg_a, msg_b, rbuf, cbuf, ebuf,
          sem_r, sem_c, sem_e, sem_g, sem_s):
        c = lax.axis_index("c")
        s = lax.axis_index("s")
        msgs = (msg_a, msg_b)
        zero = jnp.zeros((L,), jnp.float32)

        def zrow(kk, _):
            for dd in range(nd):
                msg_a[kk, pl.ds(dd * L, L)] = zero
            return 0
        lax.fori_loop(0, KB, zrow, 0)

        base = s * rows_per
        nz = rows_per // KB
        for i in range(nz):
            pltpu.sync_copy(msg_a, acc.at[pl.ds(base + i * KB, KB)])
        tz = rows_per - nz * KB
        if tz:
            pltpu.sync_copy(msg_a.at[pl.ds(0, tz)],
                            acc.at[pl.ds(base + nz * KB, tz)])
        if rows_tail:
            @pl.when(s == NS - 1)
            def _():
                pltpu.sync_copy(msg_a.at[pl.ds(0, rows_tail)],
                                acc.at[pl.ds(NS * rows_per, rows_tail)])
        plsc.subcore_barrier()

        wbase = (c * NS + s) * per_w
        wrow = wbase // KB

        def rcopy(t, x):
            return pltpu.make_async_copy(row_hbm.at[pl.ds(wbase + t * KB, KB)],
                                         rbuf.at[x], sem_r.at[x])

        def ccopy(t, x):
            return pltpu.make_async_copy(col2_hbm.at[pl.ds(wrow + t, 1)],
                                         cbuf.at[x], sem_c.at[x])

        def ecopy(t, x):
            return pltpu.make_async_copy(ew_hbm.at[pl.ds(wbase + t * KB, KB)],
                                         ebuf.at[x], sem_e.at[x])

        def gcopy(x):
            return pltpu.make_async_copy(xwp_hbm.at[rbuf.at[x]], msgs[x],
                                         sem_g.at[x])

        def scopy(x):
            return pltpu.make_async_copy(msgs[x], acc.at[cbuf.at[x, 0]],
                                         sem_s.at[x])

        def scale(x):
            m = msgs[x]

            def body(g, _):
                ew16 = ebuf[x, pl.ds(g * L, L)]
                for jj in range(L):
                    kk = g * L + jj
                    ewv = jnp.full((L,), ew16[jj], jnp.float32)
                    for dd in range(nd):
                        m[kk, pl.ds(dd * L, L)] = m[kk, pl.ds(dd * L, L)] * ewv
                return 0
            lax.fori_loop(0, KB // L, body, 0)

        for x in (0, 1):
            rcopy(x, x).start()
            ecopy(x, x).start()

        def pair(p, _):
            t0 = 2 * p

            for x in (0, 1):
                rcopy(t0 + x, x).wait()

                @pl.when(p >= 1)
                def _():
                    scopy(x).wait()
                ccopy(t0 + x, x).start()
                gcopy(x).start()

            for x in (0, 1):
                gcopy(x).wait()

                @pl.when(p < npairs - 1)
                def _():
                    rcopy(t0 + 2 + x, x).start()
                ecopy(t0 + x, x).wait()
                scale(x)
                ccopy(t0 + x, x).wait()
                scopy(x).start(add=True)

                @pl.when(p < npairs - 1)
                def _():
                    ecopy(t0 + 2 + x, x).start()
            return 0
        lax.fori_loop(0, npairs, pair, 0)
        for x in (0, 1):
            scopy(x).wait()

        plsc.subcore_barrier()
        pltpu.sync_copy(acc.at[pl.ds(base, rows_per)],
                        out_hbm.at[c, pl.ds(base, rows_per)])
        if rows_tail:
            @pl.when(s == NS - 1)
            def _():
                pltpu.sync_copy(acc.at[pl.ds(NS * rows_per, rows_tail)],
                                out_hbm.at[c, pl.ds(NS * rows_per, rows_tail)])

    return k(xwp, row, col2d, ew)


def _tc_pre(x, w0, degp, n, d):
    def body(x_ref, w_ref, degp_ref, dinv_ref, xwp_ref):
        dsum = jnp.sum(degp_ref[0], axis=1, keepdims=True) \
             + jnp.sum(degp_ref[1], axis=1, keepdims=True)
        deg = 1.0 + dsum * (1.0 / L)
        dinv = lax.rsqrt(deg)
        dinv_ref[...] = dinv
        xw = lax.dot_general(x_ref[...], w_ref[...], (((1,), (1,)), ((), ())),
                             preferred_element_type=jnp.float32)
        xwp_ref[...] = xw * dinv

    return pl.pallas_call(
        body,
        out_shape=(jax.ShapeDtypeStruct((n, 1), jnp.float32),
                   jax.ShapeDtypeStruct((n, d), jnp.float32)),
    )(x, w0, degp)


def _tc_post(sp, xwp, dinv, b, gamma, beta, w_next, n, d):
    has_next = w_next is not None

    def body(*refs):
        if has_next:
            sp_ref, xwp_ref, dinv_ref, b_ref, g_ref, be_ref, wn_ref, out_ref = refs
        else:
            sp_ref, xwp_ref, dinv_ref, b_ref, g_ref, be_ref, out_ref = refs
        dinv = dinv_ref[...]
        pre = dinv * (sp_ref[0] + sp_ref[1] + xwp_ref[...]) + b_ref[...]
        mean = jnp.mean(pre, axis=0, keepdims=True)
        var = jnp.mean((pre - mean) ** 2, axis=0, keepdims=True)
        h = (pre - mean) * lax.rsqrt(var + 1e-5) * g_ref[...] + be_ref[...]
        h = jnp.maximum(h, 0.0)
        if has_next:
            xw = lax.dot_general(h, wn_ref[...], (((1,), (1,)), ((), ())),
                                 preferred_element_type=jnp.float32)
            out_ref[...] = xw * dinv
        else:
            out_ref[...] = h

    args = [sp, xwp, dinv, b, gamma, beta]
    if has_next:
        args.append(w_next)
    return pl.pallas_call(
        body,
        out_shape=jax.ShapeDtypeStruct((n, d), jnp.float32),
    )(*args)


def kernel(node_features, edge_index, edges_weight,
           W0, b0, gamma0, beta0, W1, b1, gamma1, beta1, W2, b2, gamma2, beta2):
    n, d = node_features.shape
    e = edge_index.shape[1]
    block = NC * NS * 2 * KB
    ep = -(-e // block) * block
    pad = ep - e
    row = jnp.concatenate([edge_index[0], jnp.zeros((pad,), edge_index.dtype)])
    col = jnp.concatenate([edge_index[1], jnp.zeros((pad,), edge_index.dtype)])
    ew = jnp.concatenate([edges_weight,
                          jnp.zeros((pad,), edges_weight.dtype)])
    col2d = col.reshape(ep // KB, KB)

    degp = _sc_deg(col, ew, n)
    dinv, xwp = _tc_pre(node_features, W0, degp, n, d)

    params = [(b0, gamma0, beta0, W1), (b1, gamma1, beta1, W2),
              (b2, gamma2, beta2, None)]
    for b, g, be, w_next in params:
        sp = _sc_aggregate(xwp, row, col2d, ew, n, d)
        xwp = _tc_post(sp, xwp, dinv, b, g, be, w_next, n, d)
    return xwp
